# baseline (speedup 1.0000x reference)
import functools

import jax
import jax.numpy as jnp
from jax import lax
from jax.experimental import pallas as pl
from jax.experimental.pallas import tpu as pltpu
from jax.experimental.pallas import tpu_sc as plsc

_L = 16
_NC = 2
_NS = 16
_NW = _NC * _NS
_BLK = 128


def _make_sc_nn(n, k):
    qpw = n // _NW
    ng = qpw // _L
    nb = k // _BLK
    mesh = plsc.VectorSubcoreMesh(core_axis_name="c", subcore_axis_name="s")

    @functools.partial(
        pl.kernel,
        out_type=[
            jax.ShapeDtypeStruct((n,), jnp.float32),
            jax.ShapeDtypeStruct((n,), jnp.float32),
        ],
        mesh=mesh,
        compiler_params=pltpu.CompilerParams(needs_layout_passes=False),
        scratch_types=[
            pltpu.VMEM((2 * k,), jnp.float32),
            pltpu.VMEM((2 * k,), jnp.float32),
            pltpu.VMEM((k,), jnp.float32),
            pltpu.VMEM((k,), jnp.float32),
            pltpu.VMEM((k,), jnp.float32),
            pltpu.VMEM((k,), jnp.float32),
            pltpu.VMEM((2 * qpw,), jnp.float32),
            pltpu.VMEM((2 * qpw,), jnp.float32),
            pltpu.VMEM((qpw,), jnp.float32),
            pltpu.VMEM((qpw,), jnp.float32),
            pltpu.SemaphoreType.DMA,
            pltpu.SemaphoreType.DMA,
            pltpu.SemaphoreType.DMA,
        ],
    )
    def sc_nn(of_h, nf_h, rf_h, pf_h, bf_h,
              d2_h, mk_h,
              rf_v, pf_v, bf_v, a_v, rx2_v, ry2_v, q2_v, n2_v, od_v, om_v,
              sem_p, sem_b, sem_n):
        wid = lax.axis_index("s") * _NC + lax.axis_index("c")
        base = wid * qpw
        h_pf = pltpu.make_async_copy(pf_h, pf_v, sem_p)
        h_pf.start()
        h_bf = pltpu.make_async_copy(bf_h, bf_v, sem_b)
        h_bf.start()
        h_n2 = pltpu.make_async_copy(nf_h.at[pl.ds(2 * base, 2 * qpw)], n2_v, sem_n)
        h_n2.start()
        pltpu.sync_copy(rf_h, rf_v)
        pltpu.sync_copy(of_h.at[pl.ds(2 * base, 2 * qpw)], q2_v)

        iota2 = lax.iota(jnp.int32, _L) * 2

        def pre(i, _):
            idx = iota2 + i * (2 * _L)
            rx = plsc.load_gather(rf_v, [idx])
            ry = plsc.load_gather(rf_v, [idx + 1])
            a_v[pl.ds(i * _L, _L)] = rx * rx + ry * ry
            rx2_v[pl.ds(i * _L, _L)] = rx + rx
            ry2_v[pl.ds(i * _L, _L)] = ry + ry
            return 0

        lax.fori_loop(0, k // _L, pre, 0)

        qx = [plsc.load_gather(q2_v, [iota2 + g * (2 * _L)]) for g in range(ng)]
        qy = [plsc.load_gather(q2_v, [iota2 + g * (2 * _L) + 1]) for g in range(ng)]
        inf = jnp.full((_L,), jnp.inf, jnp.float32)
        zeroi = jnp.zeros((_L,), jnp.int32)

        def blk_step(b, carry):
            st = list(carry)
            kv0 = st[2 * ng]

            def inner(_, c2):
                s2 = list(c2)
                kv = s2[ng]
                ab = plsc.load_gather(a_v, [kv])
                xb = plsc.load_gather(rx2_v, [kv])
                yb = plsc.load_gather(ry2_v, [kv])
                for g in range(ng):
                    t = ab - xb * qx[g] - yb * qy[g]
                    s2[g] = jnp.minimum(s2[g], t)
                s2[ng] = kv + 1
                return tuple(s2)

            fin2 = lax.fori_loop(0, _BLK, inner, tuple([inf] * ng + [kv0]))
            bv = jnp.full((_L,), b, jnp.int32)
            for g in range(ng):
                bb = fin2[g]
                pred = bb < st[g]
                st[g] = jnp.minimum(st[g], bb)
                st[ng + g] = jnp.where(pred, bv, st[ng + g])
            st[2 * ng] = fin2[ng]
            return tuple(st)

        fin = lax.fori_loop(0, nb, blk_step,
                            tuple([inf] * ng + [zeroi] * ng + [zeroi]))

        bigi = jnp.full((_L,), k, jnp.int32)
        h_pf.wait()
        h_bf.wait()
        h_n2.wait()
        for g in range(ng):
            bt = fin[g]
            jv0 = fin[ng + g] * _BLK

            def rescan(_, c2, g=g, bt=bt):
                cand, jv = c2
                ab = plsc.load_gather(a_v, [jv])
                xb = plsc.load_gather(rx2_v, [jv])
                yb = plsc.load_gather(ry2_v, [jv])
                t = ab - xb * qx[g] - yb * qy[g]
                cand = jnp.minimum(cand, jnp.where(t == bt, jv, bigi))
                return cand, jv + 1

            cand, _ = lax.fori_loop(0, _BLK, rescan, (bigi, jv0))
            bid = jnp.minimum(cand, k - 1)
            bid2 = bid + bid
            pxg = plsc.load_gather(pf_v, [bid2])
            pyg = plsc.load_gather(pf_v, [bid2 + 1])
            bfg = plsc.load_gather(bf_v, [bid])
            nxg = plsc.load_gather(n2_v, [iota2 + g * (2 * _L)])
            nyg = plsc.load_gather(n2_v, [iota2 + g * (2 * _L) + 1])
            ddx = nxg - pxg
            ddy = nyg - pyg
            od_v[pl.ds(g * _L, _L)] = ddx * ddx + ddy * ddy
            om_v[pl.ds(g * _L, _L)] = bfg
        pltpu.sync_copy(od_v, d2_h.at[pl.ds(base, qpw)])
        pltpu.sync_copy(om_v, mk_h.at[pl.ds(base, qpw)])

    return sc_nn


def _finish_body(d2_ref, mk_ref, out_ref):
    d = jnp.sqrt(d2_ref[...])
    m = mk_ref[...]
    out_ref[0, 0] = jnp.sum(d * m) / jnp.sum(m)


def _make_finish():
    return pl.pallas_call(
        _finish_body,
        out_shape=jax.ShapeDtypeStruct((1, 1), jnp.float32),
        out_specs=pl.BlockSpec(memory_space=pltpu.SMEM),
    )


def kernel(flat_origin_curves, flat_new_curves, dict_points, dict_ref, dict_bool):
    n = flat_origin_curves.shape[0]
    k = dict_ref.shape[0]
    of = flat_origin_curves.reshape(-1)
    nf = flat_new_curves.reshape(-1)
    rf = dict_ref.reshape(-1)
    pf = dict_points.reshape(-1)
    bf = dict_bool.astype(jnp.float32)
    d2, mk = _make_sc_nn(n, k)(of, nf, rf, pf, bf)
    loss = _make_finish()(d2.reshape(n // 128, 128), mk.reshape(n // 128, 128))
    return loss[0, 0]


# speedup vs baseline: 1.0329x; 1.0329x over previous
import functools

import jax
import jax.numpy as jnp
from jax import lax
from jax.experimental import pallas as pl
from jax.experimental.pallas import tpu as pltpu
from jax.experimental.pallas import tpu_sc as plsc

_L = 16
_NC = 2
_NS = 16
_NW = _NC * _NS


def _make_sc_nn(n, k):
    qpw = n // _NW
    ng = qpw // _L
    mesh = plsc.VectorSubcoreMesh(core_axis_name="c", subcore_axis_name="s")

    @functools.partial(
        pl.kernel,
        out_type=[
            jax.ShapeDtypeStruct((n,), jnp.float32),
            jax.ShapeDtypeStruct((n,), jnp.float32),
        ],
        mesh=mesh,
        compiler_params=pltpu.CompilerParams(needs_layout_passes=False),
        scratch_types=[
            pltpu.VMEM((k,), jnp.float32),
            pltpu.VMEM((k,), jnp.float32),
            pltpu.VMEM((k,), jnp.float32),
            pltpu.VMEM((k,), jnp.float32),
            pltpu.VMEM((k,), jnp.float32),
            pltpu.VMEM((qpw,), jnp.float32),
            pltpu.VMEM((qpw,), jnp.float32),
            pltpu.VMEM((qpw,), jnp.float32),
            pltpu.VMEM((qpw,), jnp.float32),
            pltpu.VMEM((qpw,), jnp.float32),
            pltpu.VMEM((qpw,), jnp.float32),
            pltpu.SemaphoreType.DMA,
            pltpu.SemaphoreType.DMA,
            pltpu.SemaphoreType.DMA,
            pltpu.SemaphoreType.DMA,
            pltpu.SemaphoreType.DMA,
        ],
    )
    def sc_nn(ox_h, oy_h, nx_h, ny_h, rx_h, ry_h, px_h, py_h, bf_h,
              d2_h, mk_h,
              rx_v, ry_v, px_v, py_v, bf_v,
              qx_v, qy_v, nx_v, ny_v, od_v, om_v,
              sem_px, sem_py, sem_bf, sem_nx, sem_ny):
        wid = lax.axis_index("s") * _NC + lax.axis_index("c")
        base = wid * qpw
        h_px = pltpu.make_async_copy(px_h, px_v, sem_px)
        h_px.start()
        h_py = pltpu.make_async_copy(py_h, py_v, sem_py)
        h_py.start()
        h_bf = pltpu.make_async_copy(bf_h, bf_v, sem_bf)
        h_bf.start()
        h_nx = pltpu.make_async_copy(nx_h.at[pl.ds(base, qpw)], nx_v, sem_nx)
        h_nx.start()
        h_ny = pltpu.make_async_copy(ny_h.at[pl.ds(base, qpw)], ny_v, sem_ny)
        h_ny.start()
        pltpu.sync_copy(rx_h, rx_v)
        pltpu.sync_copy(ry_h, ry_v)
        pltpu.sync_copy(ox_h.at[pl.ds(base, qpw)], qx_v)
        pltpu.sync_copy(oy_h.at[pl.ds(base, qpw)], qy_v)

        qx = [qx_v[pl.ds(g * _L, _L)] for g in range(ng)]
        qy = [qy_v[pl.ds(g * _L, _L)] for g in range(ng)]
        inf = jnp.full((_L,), jnp.inf, jnp.float32)
        zero = jnp.zeros((_L,), jnp.int32)
        init = tuple([inf] * ng + [zero] * ng + [zero])

        def step(_, carry):
            st = list(carry)
            kv = st[2 * ng]
            rxb = plsc.load_gather(rx_v, [kv])
            ryb = plsc.load_gather(ry_v, [kv])
            for g in range(ng):
                dx = rxb - qx[g]
                dy = ryb - qy[g]
                d2 = dx * dx + dy * dy
                pred = d2 < st[g]
                st[g] = jnp.where(pred, d2, st[g])
                st[ng + g] = jnp.where(pred, kv, st[ng + g])
            st[2 * ng] = kv + 1
            return tuple(st)

        fin = lax.fori_loop(0, k, step, init, unroll=2)
        h_px.wait()
        h_py.wait()
        h_bf.wait()
        h_nx.wait()
        h_ny.wait()
        for g in range(ng):
            bid = fin[ng + g]
            pxg = plsc.load_gather(px_v, [bid])
            pyg = plsc.load_gather(py_v, [bid])
            bfg = plsc.load_gather(bf_v, [bid])
            ddx = nx_v[pl.ds(g * _L, _L)] - pxg
            ddy = ny_v[pl.ds(g * _L, _L)] - pyg
            od_v[pl.ds(g * _L, _L)] = ddx * ddx + ddy * ddy
            om_v[pl.ds(g * _L, _L)] = bfg
        pltpu.sync_copy(od_v, d2_h.at[pl.ds(base, qpw)])
        pltpu.sync_copy(om_v, mk_h.at[pl.ds(base, qpw)])

    return sc_nn


def _finish_body(d2_ref, mk_ref, out_ref):
    d = jnp.sqrt(d2_ref[...])
    m = mk_ref[...]
    out_ref[0, 0] = jnp.sum(d * m) / jnp.sum(m)


def _make_finish():
    return pl.pallas_call(
        _finish_body,
        out_shape=jax.ShapeDtypeStruct((1, 1), jnp.float32),
        out_specs=pl.BlockSpec(memory_space=pltpu.SMEM),
    )


def kernel(flat_origin_curves, flat_new_curves, dict_points, dict_ref, dict_bool):
    n = flat_origin_curves.shape[0]
    k = dict_ref.shape[0]
    ox = flat_origin_curves[:, 0]
    oy = flat_origin_curves[:, 1]
    nx = flat_new_curves[:, 0]
    ny = flat_new_curves[:, 1]
    rx = dict_ref[:, 0]
    ry = dict_ref[:, 1]
    px = dict_points[:, 0]
    py = dict_points[:, 1]
    bf = dict_bool.astype(jnp.float32)
    d2, mk = _make_sc_nn(n, k)(ox, oy, nx, ny, rx, ry, px, py, bf)
    loss = _make_finish()(d2.reshape(n // 128, 128), mk.reshape(n // 128, 128))
    return loss[0, 0]
